# baseline (speedup 1.0000x reference)
import functools

import jax
import jax.numpy as jnp
from jax import lax
from jax.experimental import pallas as pl
from jax.experimental.pallas import tpu as pltpu
from jax.experimental.pallas import tpu_sc as plsc

VOCAB = 100000
EMBED = 128

_TBL_BLK = 4000
_PROJ_BLK = 4096
_GATHER_CHUNK = 128
_UNIT = 32 * _GATHER_CHUNK
_CHUNKS = ((0, 2), (2, 12), (14, 12), (26, 12), (38, 12))


def _fuse_tables_body(c_ref, d_ref, r_ref, w_ref, b_ref, t_ref):
    w = w_ref[...]
    w1 = w[:, :EMBED]
    w2 = w[:, EMBED:]
    dn = (((1,), (1,)), ((), ()))
    acc = lax.dot_general(c_ref[...], (w1 + w2) * 0.25, dn,
                          preferred_element_type=jnp.float32)
    acc += lax.dot_general(d_ref[...], w1 * 0.5, dn,
                           preferred_element_type=jnp.float32)
    acc += lax.dot_general(r_ref[...], w2 * 0.5, dn,
                           preferred_element_type=jnp.float32)
    t_ref[...] = acc + b_ref[...]


def _fuse_tables(c, d, r, w, b2d):
    n_blk = VOCAB // _TBL_BLK
    tbl_spec = pl.BlockSpec((_TBL_BLK, EMBED), lambda i: (i, 0))
    return pl.pallas_call(
        _fuse_tables_body,
        grid=(n_blk,),
        in_specs=[tbl_spec, tbl_spec, tbl_spec,
                  pl.BlockSpec((EMBED, 2 * EMBED), lambda i: (0, 0)),
                  pl.BlockSpec((1, EMBED), lambda i: (0, 0))],
        out_specs=tbl_spec,
        out_shape=jax.ShapeDtypeStruct((VOCAB, EMBED), jnp.float32),
        name="fuse_tables",
    )(c, d, r, w, b2d)


def _proj_add_body(x_ref, g_ref, w_ref, o_ref):
    w = w_ref[...]
    ws = (w[:, :EMBED] + w[:, EMBED:]) * 0.25
    dn = (((1,), (1,)), ((), ()))
    o_ref[...] = lax.dot_general(x_ref[...], ws, dn,
                                 preferred_element_type=jnp.float32
                                 ) + g_ref[...]


def _proj_add_chunk(u0, nu, x, g, w, prev_out):
    n = x.shape[0]
    blocks_per_unit = _UNIT // _PROJ_BLK
    steps = nu * blocks_per_unit
    off = u0 * blocks_per_unit
    row_spec = pl.BlockSpec((_PROJ_BLK, EMBED), lambda i: (off + i, 0))
    operands = [x, g, w]
    in_specs = [row_spec, row_spec,
                pl.BlockSpec((EMBED, 2 * EMBED), lambda i: (0, 0))]
    aliases = {}
    if prev_out is not None:
        operands.append(prev_out)
        in_specs.append(pl.BlockSpec(memory_space=pl.ANY))
        aliases = {3: 0}

    def body(x_ref, g_ref, w_ref, *rest):
        _proj_add_body(x_ref, g_ref, w_ref, rest[-1])

    return pl.pallas_call(
        body,
        grid=(steps,),
        in_specs=in_specs,
        out_specs=row_spec,
        out_shape=jax.ShapeDtypeStruct((n, EMBED), jnp.float32),
        input_output_aliases=aliases,
        name=f"proj_add_{u0}",
    )(*operands)


@functools.cache
def _make_gather(n_rows, u0, nu):
    info = plsc.get_sparse_core_info()
    nc, ns = info.num_cores, info.num_subcores
    nw = nc * ns
    units = n_rows // _UNIT
    mesh = plsc.VectorSubcoreMesh(core_axis_name="c", subcore_axis_name="s")

    @functools.partial(
        pl.kernel,
        mesh=mesh,
        out_type=jax.ShapeDtypeStruct((n_rows, EMBED), jnp.float32),
        scratch_types=[
            pltpu.VMEM((units, _GATHER_CHUNK), jnp.int32),
            pltpu.VMEM((_GATHER_CHUNK, EMBED), jnp.float32),
            pltpu.VMEM((_GATHER_CHUNK, EMBED), jnp.float32),
            pltpu.SemaphoreType.DMA,
            pltpu.SemaphoreType.DMA,
        ],
        name=f"sc_gather_{u0}",
    )
    def gather(t_hbm, idx_hbm, out_hbm, idx_v, rows0, rows1, sem0, sem1):
        wid = lax.axis_index("s") * nc + lax.axis_index("c")
        pltpu.sync_copy(idx_hbm.at[wid], idx_v)

        pltpu.async_copy(t_hbm.at[idx_v.at[u0]], rows0, sem0)

        def step(p, carry):
            j0 = u0 + p * 2
            out0 = j0 * _UNIT + wid * _GATHER_CHUNK
            pltpu.make_async_copy(t_hbm.at[idx_v.at[j0]], rows0, sem0).wait()
            pltpu.async_copy(t_hbm.at[idx_v.at[j0 + 1]], rows1, sem1)
            pltpu.sync_copy(
                rows0, out_hbm.at[pl.ds(out0, _GATHER_CHUNK)])
            pltpu.make_async_copy(t_hbm.at[idx_v.at[j0 + 1]], rows1,
                                  sem1).wait()

            @pl.when(j0 + 2 < u0 + nu)
            def _prefetch():
                pltpu.async_copy(t_hbm.at[idx_v.at[j0 + 2]], rows0, sem0)

            pltpu.sync_copy(
                rows1, out_hbm.at[pl.ds(out0 + _UNIT, _GATHER_CHUNK)])
            return carry

        lax.fori_loop(0, nu // 2, step, 0)

    return gather


def kernel(word_embeddings, texts, common_tbl, demo_tbl, rep_tbl, W, b):
    bsz, seq, emb = word_embeddings.shape
    n = bsz * seq
    info = plsc.get_sparse_core_info()
    nw = info.num_cores * info.num_subcores
    units = n // _UNIT
    idx3d = texts.reshape(units, nw, n // (units * nw)).transpose(1, 0, 2)
    idx3d = idx3d.astype(jnp.int32)
    fused_tbl = _fuse_tables(common_tbl, demo_tbl, rep_tbl, W,
                             b.reshape(1, emb))
    gathered = [_make_gather(n, u0, nu)(fused_tbl, idx3d)
                for (u0, nu) in _CHUNKS]
    we_flat = word_embeddings.reshape(n, emb)
    out = None
    for g, (u0, nu) in zip(gathered, _CHUNKS):
        out = _proj_add_chunk(u0, nu, we_flat, g, W, out)
    return out.reshape(bsz, seq, emb)


# speedup vs baseline: 1.0054x; 1.0054x over previous
import functools

import jax
import jax.numpy as jnp
from jax import lax
from jax.experimental import pallas as pl
from jax.experimental.pallas import tpu as pltpu
from jax.experimental.pallas import tpu_sc as plsc

VOCAB = 100000
EMBED = 128

_TBL_BLK = 4000
_PROJ_BLK = 4096
_GATHER_CHUNK = 128
_UNIT = 32 * _GATHER_CHUNK
_CHUNKS = ((0, 10), (10, 10), (20, 10), (30, 10), (40, 10))


def _fuse_tables_body(c_ref, d_ref, r_ref, w_ref, b_ref, t_ref):
    w = w_ref[...]
    w1 = w[:, :EMBED]
    w2 = w[:, EMBED:]
    dn = (((1,), (1,)), ((), ()))
    acc = lax.dot_general(c_ref[...], (w1 + w2) * 0.25, dn,
                          preferred_element_type=jnp.float32)
    acc += lax.dot_general(d_ref[...], w1 * 0.5, dn,
                           preferred_element_type=jnp.float32)
    acc += lax.dot_general(r_ref[...], w2 * 0.5, dn,
                           preferred_element_type=jnp.float32)
    t_ref[...] = acc + b_ref[...]


def _fuse_tables(c, d, r, w, b2d):
    n_blk = VOCAB // _TBL_BLK
    tbl_spec = pl.BlockSpec((_TBL_BLK, EMBED), lambda i: (i, 0))
    return pl.pallas_call(
        _fuse_tables_body,
        grid=(n_blk,),
        in_specs=[tbl_spec, tbl_spec, tbl_spec,
                  pl.BlockSpec((EMBED, 2 * EMBED), lambda i: (0, 0)),
                  pl.BlockSpec((1, EMBED), lambda i: (0, 0))],
        out_specs=pl.BlockSpec((_TBL_BLK, EMBED), lambda i: (i, 0)),
        out_shape=jax.ShapeDtypeStruct((VOCAB, EMBED), jnp.float32),
        name="fuse_tables",
    )(c, d, r, w, b2d)


def _proj_add_body(x_ref, g_ref, w_ref, o_ref):
    w = w_ref[...]
    ws = (w[:, :EMBED] + w[:, EMBED:]) * 0.25
    dn = (((1,), (1,)), ((), ()))
    o_ref[...] = lax.dot_general(x_ref[...], ws, dn,
                                 preferred_element_type=jnp.float32
                                 ) + g_ref[...]


def _proj_add_chunk(u0, nu, x, g, w, prev_out):
    n = x.shape[0]
    blocks_per_unit = _UNIT // _PROJ_BLK
    steps = nu * blocks_per_unit
    off = u0 * blocks_per_unit
    row_spec = pl.BlockSpec((_PROJ_BLK, EMBED), lambda i: (off + i, 0))
    operands = [x, g, w]
    in_specs = [row_spec, row_spec,
                pl.BlockSpec((EMBED, 2 * EMBED), lambda i: (0, 0))]
    aliases = {}
    if prev_out is not None:
        operands.append(prev_out)
        in_specs.append(pl.BlockSpec(memory_space=pl.ANY))
        aliases = {3: 0}

    def body(x_ref, g_ref, w_ref, *rest):
        _proj_add_body(x_ref, g_ref, w_ref, rest[-1])

    return pl.pallas_call(
        body,
        grid=(steps,),
        in_specs=in_specs,
        out_specs=row_spec,
        out_shape=jax.ShapeDtypeStruct((n, EMBED), jnp.float32),
        input_output_aliases=aliases,
        name=f"proj_add_{u0}",
    )(*operands)


@functools.cache
def _make_gather(n_rows, u0, nu):
    info = plsc.get_sparse_core_info()
    nc, ns = info.num_cores, info.num_subcores
    nw = nc * ns
    units = n_rows // _UNIT
    mesh = plsc.VectorSubcoreMesh(core_axis_name="c", subcore_axis_name="s")

    @functools.partial(
        pl.kernel,
        mesh=mesh,
        out_type=jax.ShapeDtypeStruct((n_rows, EMBED), jnp.float32),
        scratch_types=[
            pltpu.VMEM((units, _GATHER_CHUNK), jnp.int32),
            pltpu.VMEM((_GATHER_CHUNK, EMBED), jnp.float32),
            pltpu.VMEM((_GATHER_CHUNK, EMBED), jnp.float32),
            pltpu.SemaphoreType.DMA,
            pltpu.SemaphoreType.DMA,
        ],
        name=f"sc_gather_{u0}",
    )
    def gather(t_hbm, idx_hbm, out_hbm, idx_v, rows0, rows1, sem0, sem1):
        wid = lax.axis_index("s") * nc + lax.axis_index("c")
        pltpu.sync_copy(idx_hbm.at[wid], idx_v)

        pltpu.async_copy(t_hbm.at[idx_v.at[u0]], rows0, sem0)

        def step(p, carry):
            j0 = u0 + p * 2
            out0 = j0 * _UNIT + wid * _GATHER_CHUNK
            pltpu.make_async_copy(t_hbm.at[idx_v.at[j0]], rows0, sem0).wait()
            pltpu.async_copy(t_hbm.at[idx_v.at[j0 + 1]], rows1, sem1)
            pltpu.sync_copy(
                rows0, out_hbm.at[pl.ds(out0, _GATHER_CHUNK)])
            pltpu.make_async_copy(t_hbm.at[idx_v.at[j0 + 1]], rows1,
                                  sem1).wait()

            @pl.when(j0 + 2 < u0 + nu)
            def _prefetch():
                pltpu.async_copy(t_hbm.at[idx_v.at[j0 + 2]], rows0, sem0)

            pltpu.sync_copy(
                rows1, out_hbm.at[pl.ds(out0 + _UNIT, _GATHER_CHUNK)])
            return carry

        lax.fori_loop(0, nu // 2, step, 0)

    return gather


def kernel(word_embeddings, texts, common_tbl, demo_tbl, rep_tbl, W, b):
    bsz, seq, emb = word_embeddings.shape
    n = bsz * seq
    info = plsc.get_sparse_core_info()
    nw = info.num_cores * info.num_subcores
    units = n // _UNIT
    idx3d = texts.reshape(units, nw, n // (units * nw)).transpose(1, 0, 2)
    idx3d = idx3d.astype(jnp.int32)
    fused_tbl = _fuse_tables(common_tbl, demo_tbl, rep_tbl, W,
                             b.reshape(1, emb))
    gathered = [_make_gather(n, u0, nu)(fused_tbl, idx3d)
                for (u0, nu) in _CHUNKS]
    we_flat = word_embeddings.reshape(n, emb)
    out = None
    for g, (u0, nu) in zip(gathered, _CHUNKS):
        out = _proj_add_chunk(u0, nu, we_flat, g, W, out)
    return out.reshape(bsz, seq, emb)


# speedup vs baseline: 1.0121x; 1.0067x over previous
import functools

import jax
import jax.numpy as jnp
from jax import lax
from jax.experimental import pallas as pl
from jax.experimental.pallas import tpu as pltpu
from jax.experimental.pallas import tpu_sc as plsc

VOCAB = 100000
EMBED = 128

_TBL_BLK = 4000
_PROJ_BLK = 4096
_GATHER_CHUNK = 128
_K = 5


def _fuse_tables_body(c_ref, d_ref, r_ref, w_ref, b_ref, t_ref):
    w = w_ref[...]
    w1 = w[:, :EMBED]
    w2 = w[:, EMBED:]
    dn = (((1,), (1,)), ((), ()))
    acc = lax.dot_general(c_ref[...], (w1 + w2) * 0.25, dn,
                          preferred_element_type=jnp.float32)
    acc += lax.dot_general(d_ref[...], w1 * 0.5, dn,
                           preferred_element_type=jnp.float32)
    acc += lax.dot_general(r_ref[...], w2 * 0.5, dn,
                           preferred_element_type=jnp.float32)
    t_ref[...] = acc + b_ref[...]


def _fuse_tables(c, d, r, w, b2d):
    n_blk = VOCAB // _TBL_BLK
    tbl_spec = pl.BlockSpec((_TBL_BLK, EMBED), lambda i: (i, 0))
    return pl.pallas_call(
        _fuse_tables_body,
        grid=(n_blk,),
        in_specs=[tbl_spec, tbl_spec, tbl_spec,
                  pl.BlockSpec((EMBED, 2 * EMBED), lambda i: (0, 0)),
                  pl.BlockSpec((1, EMBED), lambda i: (0, 0))],
        out_specs=tbl_spec,
        out_shape=jax.ShapeDtypeStruct((VOCAB, EMBED), jnp.float32),
        name="fuse_tables",
    )(c, d, r, w, b2d)


def _proj_add_body(x_ref, g_ref, w_ref, o_ref):
    w = w_ref[...]
    ws = (w[:, :EMBED] + w[:, EMBED:]) * 0.25
    dn = (((1,), (1,)), ((), ()))
    o_ref[...] = lax.dot_general(x_ref[...], ws, dn,
                                 preferred_element_type=jnp.float32
                                 ) + g_ref[...]


def _proj_add_chunk(ci, x, g, w, prev_out):
    n = x.shape[0]
    steps = n // _K // _PROJ_BLK
    off = ci * steps
    row_spec = pl.BlockSpec((_PROJ_BLK, EMBED), lambda i: (off + i, 0))
    operands = [x, g, w]
    in_specs = [row_spec, row_spec,
                pl.BlockSpec((EMBED, 2 * EMBED), lambda i: (0, 0))]
    aliases = {}
    if prev_out is not None:
        operands.append(prev_out)
        in_specs.append(pl.BlockSpec(memory_space=pl.ANY))
        aliases = {3: 0}

    def body(x_ref, g_ref, w_ref, *rest):
        _proj_add_body(x_ref, g_ref, w_ref, rest[-1])

    return pl.pallas_call(
        body,
        grid=(steps,),
        in_specs=in_specs,
        out_specs=row_spec,
        out_shape=jax.ShapeDtypeStruct((n, EMBED), jnp.float32),
        input_output_aliases=aliases,
        name=f"proj_add_{ci}",
    )(*operands)


@functools.cache
def _make_gather(n_rows, ci):
    info = plsc.get_sparse_core_info()
    nc, ns = info.num_cores, info.num_subcores
    nw = nc * ns
    chunks = n_rows // _K // nw // _GATHER_CHUNK
    mesh = plsc.VectorSubcoreMesh(core_axis_name="c", subcore_axis_name="s")

    @functools.partial(
        pl.kernel,
        mesh=mesh,
        out_type=jax.ShapeDtypeStruct((n_rows, EMBED), jnp.float32),
        scratch_types=[
            pltpu.VMEM((chunks, _GATHER_CHUNK), jnp.int32),
            pltpu.VMEM((_GATHER_CHUNK, EMBED), jnp.float32),
            pltpu.VMEM((_GATHER_CHUNK, EMBED), jnp.float32),
            pltpu.SemaphoreType.DMA,
            pltpu.SemaphoreType.DMA,
            pltpu.SemaphoreType.DMA,
            pltpu.SemaphoreType.DMA,
        ],
        name=f"sc_gather_{ci}",
    )
    def gather(t_hbm, idx_hbm, out_hbm, idx_v, rows0, rows1,
               gsem0, gsem1, wsem0, wsem1):
        wid = lax.axis_index("s") * nc + lax.axis_index("c")
        pltpu.sync_copy(idx_hbm.at[ci * nw + wid], idx_v)
        base = (ci * nw + wid) * chunks * _GATHER_CHUNK

        def dst(j):
            return out_hbm.at[pl.ds(base + j * _GATHER_CHUNK, _GATHER_CHUNK)]

        pltpu.async_copy(t_hbm.at[idx_v.at[0]], rows0, gsem0)
        pltpu.make_async_copy(t_hbm.at[idx_v.at[0]], rows0, gsem0).wait()
        pltpu.async_copy(rows0, dst(0), wsem0)
        pltpu.async_copy(t_hbm.at[idx_v.at[1]], rows1, gsem1)

        def step(p, carry):
            j0 = p * 2
            pltpu.make_async_copy(t_hbm.at[idx_v.at[j0 + 1]], rows1,
                                  gsem1).wait()
            pltpu.async_copy(rows1, dst(j0 + 1), wsem1)
            pltpu.make_async_copy(rows0, dst(j0), wsem0).wait()

            @pl.when(j0 + 2 < chunks)
            def _gather_next_even():
                pltpu.async_copy(t_hbm.at[idx_v.at[j0 + 2]], rows0, gsem0)

            pltpu.make_async_copy(rows1, dst(j0 + 1), wsem1).wait()

            @pl.when(j0 + 2 < chunks)
            def _advance_odd():
                pltpu.make_async_copy(t_hbm.at[idx_v.at[j0 + 2]], rows0,
                                      gsem0).wait()
                pltpu.async_copy(rows0, dst(j0 + 2), wsem0)
                pltpu.async_copy(t_hbm.at[idx_v.at[j0 + 3]], rows1, gsem1)

            return carry

        lax.fori_loop(0, chunks // 2, step, 0)

    return gather


def kernel(word_embeddings, texts, common_tbl, demo_tbl, rep_tbl, W, b):
    bsz, seq, emb = word_embeddings.shape
    n = bsz * seq
    info = plsc.get_sparse_core_info()
    nw = info.num_cores * info.num_subcores
    idx3d = texts.reshape(_K * nw, n // (_K * nw * _GATHER_CHUNK),
                          _GATHER_CHUNK).astype(jnp.int32)
    fused_tbl = _fuse_tables(common_tbl, demo_tbl, rep_tbl, W,
                             b.reshape(1, emb))
    gathered = [_make_gather(n, ci)(fused_tbl, idx3d) for ci in range(_K)]
    we_flat = word_embeddings.reshape(n, emb)
    out = None
    for ci in range(_K):
        out = _proj_add_chunk(ci, we_flat, gathered[ci], W, out)
    return out.reshape(bsz, seq, emb)


# speedup vs baseline: 1.0270x; 1.0147x over previous
import functools

import jax
import jax.numpy as jnp
from jax import lax
from jax.experimental import pallas as pl
from jax.experimental.pallas import tpu as pltpu
from jax.experimental.pallas import tpu_sc as plsc

VOCAB = 100000
EMBED = 128

_TBL_BLK = 4000
_PROJ_BLK = 4096
_GATHER_CHUNK = 256
_K = 5


def _fuse_tables_body(c_ref, d_ref, r_ref, w_ref, b_ref, t_ref):
    w = w_ref[...]
    w1 = w[:, :EMBED]
    w2 = w[:, EMBED:]
    dn = (((1,), (1,)), ((), ()))
    acc = lax.dot_general(c_ref[...], (w1 + w2) * 0.25, dn,
                          preferred_element_type=jnp.float32)
    acc += lax.dot_general(d_ref[...], w1 * 0.5, dn,
                           preferred_element_type=jnp.float32)
    acc += lax.dot_general(r_ref[...], w2 * 0.5, dn,
                           preferred_element_type=jnp.float32)
    t_ref[...] = acc + b_ref[...]


def _fuse_tables(c, d, r, w, b2d):
    n_blk = VOCAB // _TBL_BLK
    tbl_spec = pl.BlockSpec((_TBL_BLK, EMBED), lambda i: (i, 0))
    return pl.pallas_call(
        _fuse_tables_body,
        grid=(n_blk,),
        in_specs=[tbl_spec, tbl_spec, tbl_spec,
                  pl.BlockSpec((EMBED, 2 * EMBED), lambda i: (0, 0)),
                  pl.BlockSpec((1, EMBED), lambda i: (0, 0))],
        out_specs=tbl_spec,
        out_shape=jax.ShapeDtypeStruct((VOCAB, EMBED), jnp.float32),
        name="fuse_tables",
    )(c, d, r, w, b2d)


def _proj_add_body(x_ref, g_ref, w_ref, o_ref):
    w = w_ref[...]
    ws = (w[:, :EMBED] + w[:, EMBED:]) * 0.25
    dn = (((1,), (1,)), ((), ()))
    o_ref[...] = lax.dot_general(x_ref[...], ws, dn,
                                 preferred_element_type=jnp.float32
                                 ) + g_ref[...]


def _proj_add_chunk(ci, x, g, w, prev_out):
    n = x.shape[0]
    steps = n // _K // _PROJ_BLK
    off = ci * steps
    row_spec = pl.BlockSpec((_PROJ_BLK, EMBED), lambda i: (off + i, 0))
    operands = [x, g, w]
    in_specs = [row_spec, row_spec,
                pl.BlockSpec((EMBED, 2 * EMBED), lambda i: (0, 0))]
    aliases = {}
    if prev_out is not None:
        operands.append(prev_out)
        in_specs.append(pl.BlockSpec(memory_space=pl.ANY))
        aliases = {3: 0}

    def body(x_ref, g_ref, w_ref, *rest):
        _proj_add_body(x_ref, g_ref, w_ref, rest[-1])

    return pl.pallas_call(
        body,
        grid=(steps,),
        in_specs=in_specs,
        out_specs=row_spec,
        out_shape=jax.ShapeDtypeStruct((n, EMBED), jnp.float32),
        input_output_aliases=aliases,
        name=f"proj_add_{ci}",
    )(*operands)


@functools.cache
def _make_gather(n_rows, ci):
    info = plsc.get_sparse_core_info()
    nc, ns = info.num_cores, info.num_subcores
    nw = nc * ns
    chunks = n_rows // _K // nw // _GATHER_CHUNK
    mesh = plsc.VectorSubcoreMesh(core_axis_name="c", subcore_axis_name="s")

    @functools.partial(
        pl.kernel,
        mesh=mesh,
        out_type=jax.ShapeDtypeStruct((n_rows, EMBED), jnp.float32),
        scratch_types=[
            pltpu.VMEM((chunks * _GATHER_CHUNK,), jnp.int32),
            pltpu.VMEM((_GATHER_CHUNK, EMBED), jnp.float32),
            pltpu.VMEM((_GATHER_CHUNK, EMBED), jnp.float32),
            pltpu.SemaphoreType.DMA,
            pltpu.SemaphoreType.DMA,
            pltpu.SemaphoreType.DMA,
            pltpu.SemaphoreType.DMA,
        ],
        name=f"sc_gather_{ci}",
    )
    def gather(t_hbm, idx_hbm, out_hbm, idx_v, rows0, rows1,
               gsem0, gsem1, wsem0, wsem1):
        wid = lax.axis_index("s") * nc + lax.axis_index("c")
        base = (ci * nw + wid) * chunks * _GATHER_CHUNK
        pltpu.sync_copy(idx_hbm.at[pl.ds(base, chunks * _GATHER_CHUNK)],
                        idx_v)

        def idx_at(j):
            return idx_v.at[pl.ds(j * _GATHER_CHUNK, _GATHER_CHUNK)]

        def dst(j):
            return out_hbm.at[pl.ds(base + j * _GATHER_CHUNK, _GATHER_CHUNK)]

        bufs = (rows0, rows1)
        gsems = (gsem0, gsem1)
        wsems = (wsem0, wsem1)
        pending = [None, None]
        pltpu.async_copy(t_hbm.at[idx_at(0)], rows0, gsem0)
        for j in range(chunks):
            cur = j % 2
            pltpu.make_async_copy(t_hbm.at[idx_at(j)], bufs[cur],
                                  gsems[cur]).wait()
            if j + 1 < chunks:
                nxt = (j + 1) % 2
                if pending[nxt] is not None:
                    pltpu.make_async_copy(bufs[nxt], dst(pending[nxt]),
                                          wsems[nxt]).wait()
                    pending[nxt] = None
                pltpu.async_copy(t_hbm.at[idx_at(j + 1)], bufs[nxt],
                                 gsems[nxt])
            pltpu.async_copy(bufs[cur], dst(j), wsems[cur])
            pending[cur] = j
        for par in (0, 1):
            if pending[par] is not None:
                pltpu.make_async_copy(bufs[par], dst(pending[par]),
                                      wsems[par]).wait()

    return gather


def kernel(word_embeddings, texts, common_tbl, demo_tbl, rep_tbl, W, b):
    bsz, seq, emb = word_embeddings.shape
    n = bsz * seq
    info = plsc.get_sparse_core_info()
    nw = info.num_cores * info.num_subcores
    idx_flat = texts.reshape(n).astype(jnp.int32)
    fused_tbl = _fuse_tables(common_tbl, demo_tbl, rep_tbl, W,
                             b.reshape(1, emb))
    gathered = [_make_gather(n, ci)(fused_tbl, idx_flat) for ci in range(_K)]
    we_flat = word_embeddings.reshape(n, emb)
    out = None
    for ci in range(_K):
        out = _proj_add_chunk(ci, we_flat, gathered[ci], W, out)
    return out.reshape(bsz, seq, emb)


# speedup vs baseline: 1.0282x; 1.0011x over previous
import functools

import jax
import jax.numpy as jnp
from jax import lax
from jax.experimental import pallas as pl
from jax.experimental.pallas import tpu as pltpu
from jax.experimental.pallas import tpu_sc as plsc

VOCAB = 100000
EMBED = 128

_TBL_BLK = 4000
_PROJ_BLK = 4096
_GATHER_CHUNK = 320
_K = 5


def _fuse_tables_body(c_ref, d_ref, r_ref, w_ref, b_ref, t_ref):
    w = w_ref[...]
    w1 = w[:, :EMBED]
    w2 = w[:, EMBED:]
    dn = (((1,), (1,)), ((), ()))
    acc = lax.dot_general(c_ref[...], (w1 + w2) * 0.25, dn,
                          preferred_element_type=jnp.float32)
    acc += lax.dot_general(d_ref[...], w1 * 0.5, dn,
                           preferred_element_type=jnp.float32)
    acc += lax.dot_general(r_ref[...], w2 * 0.5, dn,
                           preferred_element_type=jnp.float32)
    t_ref[...] = acc + b_ref[...]


def _fuse_tables(c, d, r, w, b2d):
    n_blk = VOCAB // _TBL_BLK
    tbl_spec = pl.BlockSpec((_TBL_BLK, EMBED), lambda i: (i, 0))
    return pl.pallas_call(
        _fuse_tables_body,
        grid=(n_blk,),
        in_specs=[tbl_spec, tbl_spec, tbl_spec,
                  pl.BlockSpec((EMBED, 2 * EMBED), lambda i: (0, 0)),
                  pl.BlockSpec((1, EMBED), lambda i: (0, 0))],
        out_specs=tbl_spec,
        out_shape=jax.ShapeDtypeStruct((VOCAB, EMBED), jnp.float32),
        name="fuse_tables",
    )(c, d, r, w, b2d)


def _proj_add_body(x_ref, g_ref, w_ref, o_ref):
    w = w_ref[...]
    ws = (w[:, :EMBED] + w[:, EMBED:]) * 0.25
    dn = (((1,), (1,)), ((), ()))
    o_ref[...] = lax.dot_general(x_ref[...], ws, dn,
                                 preferred_element_type=jnp.float32
                                 ) + g_ref[...]


def _proj_add_chunk(ci, x, g, w, prev_out):
    n = x.shape[0]
    steps = n // _K // _PROJ_BLK
    off = ci * steps
    row_spec = pl.BlockSpec((_PROJ_BLK, EMBED), lambda i: (off + i, 0))
    operands = [x, g, w]
    in_specs = [row_spec, row_spec,
                pl.BlockSpec((EMBED, 2 * EMBED), lambda i: (0, 0))]
    aliases = {}
    if prev_out is not None:
        operands.append(prev_out)
        in_specs.append(pl.BlockSpec(memory_space=pl.ANY))
        aliases = {3: 0}

    def body(x_ref, g_ref, w_ref, *rest):
        _proj_add_body(x_ref, g_ref, w_ref, rest[-1])

    return pl.pallas_call(
        body,
        grid=(steps,),
        in_specs=in_specs,
        out_specs=row_spec,
        out_shape=jax.ShapeDtypeStruct((n, EMBED), jnp.float32),
        input_output_aliases=aliases,
        name=f"proj_add_{ci}",
    )(*operands)


@functools.cache
def _make_gather(n_rows, ci):
    info = plsc.get_sparse_core_info()
    nc, ns = info.num_cores, info.num_subcores
    nw = nc * ns
    chunks = n_rows // _K // nw // _GATHER_CHUNK
    mesh = plsc.VectorSubcoreMesh(core_axis_name="c", subcore_axis_name="s")

    @functools.partial(
        pl.kernel,
        mesh=mesh,
        out_type=jax.ShapeDtypeStruct((n_rows, EMBED), jnp.float32),
        scratch_types=[
            pltpu.VMEM((chunks * _GATHER_CHUNK,), jnp.int32),
            pltpu.VMEM((_GATHER_CHUNK, EMBED), jnp.float32),
            pltpu.VMEM((_GATHER_CHUNK, EMBED), jnp.float32),
            pltpu.SemaphoreType.DMA,
            pltpu.SemaphoreType.DMA,
            pltpu.SemaphoreType.DMA,
            pltpu.SemaphoreType.DMA,
        ],
        name=f"sc_gather_{ci}",
    )
    def gather(t_hbm, idx_hbm, out_hbm, idx_v, rows0, rows1,
               gsem0, gsem1, wsem0, wsem1):
        wid = lax.axis_index("s") * nc + lax.axis_index("c")
        base = (ci * nw + wid) * chunks * _GATHER_CHUNK
        pltpu.sync_copy(idx_hbm.at[pl.ds(base, chunks * _GATHER_CHUNK)],
                        idx_v)

        def idx_at(j):
            return idx_v.at[pl.ds(j * _GATHER_CHUNK, _GATHER_CHUNK)]

        def dst(j):
            return out_hbm.at[pl.ds(base + j * _GATHER_CHUNK, _GATHER_CHUNK)]

        bufs = (rows0, rows1)
        gsems = (gsem0, gsem1)
        wsems = (wsem0, wsem1)
        pending = [None, None]
        pltpu.async_copy(t_hbm.at[idx_at(0)], rows0, gsem0)
        for j in range(chunks):
            cur = j % 2
            pltpu.make_async_copy(t_hbm.at[idx_at(j)], bufs[cur],
                                  gsems[cur]).wait()
            if j + 1 < chunks:
                nxt = (j + 1) % 2
                if pending[nxt] is not None:
                    pltpu.make_async_copy(bufs[nxt], dst(pending[nxt]),
                                          wsems[nxt]).wait()
                    pending[nxt] = None
                pltpu.async_copy(t_hbm.at[idx_at(j + 1)], bufs[nxt],
                                 gsems[nxt])
            pltpu.async_copy(bufs[cur], dst(j), wsems[cur])
            pending[cur] = j
        for par in (0, 1):
            if pending[par] is not None:
                pltpu.make_async_copy(bufs[par], dst(pending[par]),
                                      wsems[par]).wait()

    return gather


def kernel(word_embeddings, texts, common_tbl, demo_tbl, rep_tbl, W, b):
    bsz, seq, emb = word_embeddings.shape
    n = bsz * seq
    info = plsc.get_sparse_core_info()
    nw = info.num_cores * info.num_subcores
    idx_flat = texts.reshape(n).astype(jnp.int32)
    fused_tbl = _fuse_tables(common_tbl, demo_tbl, rep_tbl, W,
                             b.reshape(1, emb))
    gathered = [_make_gather(n, ci)(fused_tbl, idx_flat) for ci in range(_K)]
    we_flat = word_embeddings.reshape(n, emb)
    out = None
    for ci in range(_K):
        out = _proj_add_chunk(ci, we_flat, gathered[ci], W, out)
    return out.reshape(bsz, seq, emb)


# speedup vs baseline: 1.0391x; 1.0107x over previous
import functools

import jax
import jax.numpy as jnp
from jax import lax
from jax.experimental import pallas as pl
from jax.experimental.pallas import tpu as pltpu
from jax.experimental.pallas import tpu_sc as plsc

VOCAB = 100000
EMBED = 128

_TBL_BLK = 4000
_PROJ_BLK = 5120
_GATHER_CHUNK = 320
_CHUNK_UNITS = ((0, 1), (1, 5), (6, 5), (11, 5), (16, 4))
_UNIT_ROWS = 2 * _PROJ_BLK


def _fuse_tables_body(c_ref, d_ref, r_ref, w_ref, b_ref, t_ref):
    w = w_ref[...]
    w1 = w[:, :EMBED]
    w2 = w[:, EMBED:]
    dn = (((1,), (1,)), ((), ()))
    acc = lax.dot_general(c_ref[...], (w1 + w2) * 0.25, dn,
                          preferred_element_type=jnp.float32)
    acc += lax.dot_general(d_ref[...], w1 * 0.5, dn,
                           preferred_element_type=jnp.float32)
    acc += lax.dot_general(r_ref[...], w2 * 0.5, dn,
                           preferred_element_type=jnp.float32)
    t_ref[...] = acc + b_ref[...]


def _fuse_tables(c, d, r, w, b2d):
    n_blk = VOCAB // _TBL_BLK
    tbl_spec = pl.BlockSpec((_TBL_BLK, EMBED), lambda i: (i, 0))
    return pl.pallas_call(
        _fuse_tables_body,
        grid=(n_blk,),
        in_specs=[tbl_spec, tbl_spec, tbl_spec,
                  pl.BlockSpec((EMBED, 2 * EMBED), lambda i: (0, 0)),
                  pl.BlockSpec((1, EMBED), lambda i: (0, 0))],
        out_specs=tbl_spec,
        out_shape=jax.ShapeDtypeStruct((VOCAB, EMBED), jnp.float32),
        name="fuse_tables",
    )(c, d, r, w, b2d)


def _proj_add_body(x_ref, g_ref, w_ref, o_ref):
    w = w_ref[...]
    ws = (w[:, :EMBED] + w[:, EMBED:]) * 0.25
    dn = (((1,), (1,)), ((), ()))
    o_ref[...] = lax.dot_general(x_ref[...], ws, dn,
                                 preferred_element_type=jnp.float32
                                 ) + g_ref[...]


def _proj_add_chunk(ci, x, g, w, prev_out):
    n = x.shape[0]
    u0, ku = ci
    steps = ku * (_UNIT_ROWS // _PROJ_BLK)
    off = u0 * (_UNIT_ROWS // _PROJ_BLK)
    row_spec = pl.BlockSpec((_PROJ_BLK, EMBED), lambda i: (off + i, 0))
    operands = [x, g, w]
    in_specs = [row_spec, row_spec,
                pl.BlockSpec((EMBED, 2 * EMBED), lambda i: (0, 0))]
    aliases = {}
    if prev_out is not None:
        operands.append(prev_out)
        in_specs.append(pl.BlockSpec(memory_space=pl.ANY))
        aliases = {3: 0}

    def body(x_ref, g_ref, w_ref, *rest):
        _proj_add_body(x_ref, g_ref, w_ref, rest[-1])

    return pl.pallas_call(
        body,
        grid=(steps,),
        in_specs=in_specs,
        out_specs=row_spec,
        out_shape=jax.ShapeDtypeStruct((n, EMBED), jnp.float32),
        input_output_aliases=aliases,
        name=f"proj_add_{ci[0]}",
    )(*operands)


@functools.cache
def _make_gather(n_rows, u0, ku):
    info = plsc.get_sparse_core_info()
    nc, ns = info.num_cores, info.num_subcores
    nw = nc * ns
    chunks = ku * _UNIT_ROWS // nw // _GATHER_CHUNK
    mesh = plsc.VectorSubcoreMesh(core_axis_name="c", subcore_axis_name="s")

    @functools.partial(
        pl.kernel,
        mesh=mesh,
        out_type=jax.ShapeDtypeStruct((n_rows, EMBED), jnp.float32),
        scratch_types=[
            pltpu.VMEM((chunks * _GATHER_CHUNK,), jnp.int32),
            pltpu.VMEM((_GATHER_CHUNK, EMBED), jnp.float32),
            pltpu.VMEM((_GATHER_CHUNK, EMBED), jnp.float32),
            pltpu.SemaphoreType.DMA,
            pltpu.SemaphoreType.DMA,
            pltpu.SemaphoreType.DMA,
            pltpu.SemaphoreType.DMA,
        ],
        name=f"sc_gather_{u0}",
    )
    def gather(t_hbm, idx_hbm, out_hbm, idx_v, rows0, rows1,
               gsem0, gsem1, wsem0, wsem1):
        wid = lax.axis_index("s") * nc + lax.axis_index("c")
        base = u0 * _UNIT_ROWS + wid * chunks * _GATHER_CHUNK
        pltpu.sync_copy(idx_hbm.at[pl.ds(base, chunks * _GATHER_CHUNK)],
                        idx_v)

        def idx_at(j):
            return idx_v.at[pl.ds(j * _GATHER_CHUNK, _GATHER_CHUNK)]

        def dst(j):
            return out_hbm.at[pl.ds(base + j * _GATHER_CHUNK, _GATHER_CHUNK)]

        bufs = (rows0, rows1)
        gsems = (gsem0, gsem1)
        wsems = (wsem0, wsem1)
        pending = [None, None]
        pltpu.async_copy(t_hbm.at[idx_at(0)], rows0, gsem0)
        for j in range(chunks):
            cur = j % 2
            pltpu.make_async_copy(t_hbm.at[idx_at(j)], bufs[cur],
                                  gsems[cur]).wait()
            if j + 1 < chunks:
                nxt = (j + 1) % 2
                if pending[nxt] is not None:
                    pltpu.make_async_copy(bufs[nxt], dst(pending[nxt]),
                                          wsems[nxt]).wait()
                    pending[nxt] = None
                pltpu.async_copy(t_hbm.at[idx_at(j + 1)], bufs[nxt],
                                 gsems[nxt])
            pltpu.async_copy(bufs[cur], dst(j), wsems[cur])
            pending[cur] = j
        for par in (0, 1):
            if pending[par] is not None:
                pltpu.make_async_copy(bufs[par], dst(pending[par]),
                                      wsems[par]).wait()

    return gather


def kernel(word_embeddings, texts, common_tbl, demo_tbl, rep_tbl, W, b):
    bsz, seq, emb = word_embeddings.shape
    n = bsz * seq
    info = plsc.get_sparse_core_info()
    nw = info.num_cores * info.num_subcores
    idx_flat = texts.reshape(n).astype(jnp.int32)
    fused_tbl = _fuse_tables(common_tbl, demo_tbl, rep_tbl, W,
                             b.reshape(1, emb))
    gathered = [_make_gather(n, u0, ku)(fused_tbl, idx_flat)
                for (u0, ku) in _CHUNK_UNITS]
    we_flat = word_embeddings.reshape(n, emb)
    out = None
    for g, cu in zip(gathered, _CHUNK_UNITS):
        out = _proj_add_chunk(cu, we_flat, g, W, out)
    return out.reshape(bsz, seq, emb)
